# Initial kernel scaffold; baseline (speedup 1.0000x reference)
import jax
import jax.numpy as jnp
from jax.experimental import pallas as pl


def kernel(mem, idx, W):
    raise NotImplementedError("write your pallas kernel here")


# speedup vs baseline: 2.7134x; 2.7134x over previous
import functools

import jax
import jax.numpy as jnp
from jax import lax
from jax.experimental import pallas as pl
from jax.experimental.pallas import tpu as pltpu
from jax.experimental.pallas import tpu_sc as plsc

_T = 10
_CHUNK = 128


def _sc_info():
    info = plsc.get_sparse_core_info()
    return info.num_cores, info.num_subcores


def _chain_body(x_ref, w_ref, o_ref):
    w = w_ref[...]

    def step(_, x):
        y = jax.lax.dot(x, w, precision=jax.lax.Precision.HIGHEST,
                        preferred_element_type=jnp.float32)
        return x + 0.1 * jnp.tanh(y)

    o_ref[...] = jax.lax.fori_loop(0, _T, step, x_ref[...])


def _tc_chain(x, W):
    B, D = x.shape
    blk = min(B, 2048)
    return pl.pallas_call(
        _chain_body,
        grid=(B // blk,),
        in_specs=[
            pl.BlockSpec((blk, D), lambda i: (i, 0)),
            pl.BlockSpec((D, D), lambda i: (0, 0)),
        ],
        out_specs=pl.BlockSpec((blk, D), lambda i: (i, 0)),
        out_shape=jax.ShapeDtypeStruct((B, D), x.dtype),
    )(x, W)


def _make_sc_gather(M, D, B, NC, NS):
    NW = NC * NS
    b_per_w = B // NW
    n_chunks = b_per_w // _CHUNK
    mesh = plsc.VectorSubcoreMesh(core_axis_name="c", subcore_axis_name="s")

    @functools.partial(
        pl.kernel, mesh=mesh,
        out_type=jax.ShapeDtypeStruct((B, D), jnp.float32),
        compiler_params=pltpu.CompilerParams(use_tc_tiling_on_sc=False),
        scratch_types=[
            pltpu.VMEM((n_chunks, _CHUNK), jnp.int32),
            pltpu.VMEM((b_per_w, D), jnp.float32),
            pltpu.SemaphoreType.DMA,
        ],
    )
    def k(mem_hbm, idx_hbm, out_hbm, idx_v, rows_v, sem):
        wid = lax.axis_index("s") * NC + lax.axis_index("c")
        base = wid * b_per_w
        pltpu.sync_copy(idx_hbm.at[wid], idx_v)
        descs = [
            pltpu.async_copy(mem_hbm.at[idx_v.at[j]],
                             rows_v.at[pl.ds(j * _CHUNK, _CHUNK)], sem)
            for j in range(n_chunks)
        ]
        for d_ in descs:
            d_.wait()
        pltpu.sync_copy(rows_v, out_hbm.at[pl.ds(base, b_per_w)])

    return k


def _make_sc_scatter(M, D, B, NC, NS):
    NW = NC * NS
    b_per_w = B // NW
    n_chunks = b_per_w // _CHUNK
    mesh = plsc.VectorSubcoreMesh(core_axis_name="c", subcore_axis_name="s")

    @functools.partial(
        pl.kernel, mesh=mesh,
        out_type=(),
        compiler_params=pltpu.CompilerParams(use_tc_tiling_on_sc=False),
        scratch_types=[
            pltpu.VMEM((n_chunks, _CHUNK), jnp.int32),
            pltpu.VMEM((b_per_w, D), jnp.float32),
            pltpu.SemaphoreType.DMA,
        ],
    )
    def k(idx_hbm, x_hbm, buf_hbm, idx_v, rows_v, sem):
        wid = lax.axis_index("s") * NC + lax.axis_index("c")
        base = wid * b_per_w
        pltpu.sync_copy(idx_hbm.at[wid], idx_v)
        pltpu.sync_copy(x_hbm.at[pl.ds(base, b_per_w)], rows_v)
        descs = [
            pltpu.async_copy(rows_v.at[pl.ds(j * _CHUNK, _CHUNK)],
                             buf_hbm.at[idx_v.at[j]], sem)
            for j in range(n_chunks)
        ]
        for d_ in descs:
            d_.wait()

    return k


def kernel(mem, idx, W):
    M, D = mem.shape
    B = idx.shape[0]
    NC, NS = _sc_info()
    NW = NC * NS
    idx3 = idx.reshape(NW, (B // NW) // _CHUNK, _CHUNK)

    x = _make_sc_gather(M, D, B, NC, NS)(mem, idx3)
    x = _tc_chain(x, W)

    buf = jax.new_ref(mem)
    _make_sc_scatter(M, D, B, NC, NS)(idx3, x, buf)
    return buf[...]
